# baseline (speedup 1.0000x reference)
import functools

import jax
import jax.numpy as jnp
from jax import lax
from jax.experimental import pallas as pl
from jax.experimental.pallas import tpu as pltpu
from jax.experimental.pallas import tpu_sc as plsc

NUM_CLASSES = 1000000
EMBED_DIM = 32
BATCH = 16384
FIELDS = 26

_B = BATCH * FIELDS
_NC = 2
_NS = 16
_NW = _NC * _NS
_PER_W = _B // _NW
_CHUNK = 128
_NCHUNK = _PER_W // _CHUNK
_ROWS_PER_LINE = 128 // EMBED_DIM
_OLINES = _CHUNK // _ROWS_PER_LINE
assert _PER_W * _NW == _B and _NCHUNK * _CHUNK == _PER_W
assert _NCHUNK % 2 == 0


def _gather_body(idx_hbm, table_hbm, out_hbm,
                 idxs, lines, line0, line1, out0, out1,
                 gsem0, gsem1, ssem0, ssem1):
    linebuf = (line0, line1)
    outbuf = (out0, out1)
    gsem = (gsem0, gsem1)
    ssem = (ssem0, ssem1)

    wid = lax.axis_index("s") * _NC + lax.axis_index("c")
    base_row = wid * _NCHUNK
    obase = wid * (_PER_W // _ROWS_PER_LINE)

    pltpu.sync_copy(idx_hbm.at[pl.ds(base_row, _NCHUNK), :], idxs)

    def conv(r, _):
        for g in range(8):
            v = idxs[r, pl.ds(g * 16, 16)]
            lines[r, pl.ds(g * 16, 16)] = v >> 2
        return 0

    lax.fori_loop(0, _NCHUNK, conv, 0)

    def g_start(i, b):
        pltpu.make_async_copy(
            table_hbm.at[lines.at[i]], linebuf[b], gsem[b]
        ).start()

    def g_wait(b):
        pltpu.make_async_copy(
            table_hbm.at[lines.at[0]], linebuf[b], gsem[b]
        ).wait()

    def s_start(i, b):
        pltpu.make_async_copy(
            outbuf[b], out_hbm.at[pl.ds(obase + i * _OLINES, _OLINES)], ssem[b]
        ).start()

    def s_wait(b):
        pltpu.make_async_copy(
            outbuf[b], out_hbm.at[pl.ds(obase, _OLINES)], ssem[b]
        ).wait()

    iota = lax.iota(jnp.int32, 16)

    def compact(i, b):
        for g in range(_CHUNK // 16):
            r0 = g * 16
            rowv = iota + r0
            qrow = rowv >> 2
            dcol0 = (rowv & 3) * 32
            idxv = idxs[i, pl.ds(r0, 16)]
            scol0 = (idxv & 3) * 32
            for c in range(EMBED_DIM):
                vals = plsc.load_gather(linebuf[b], [rowv, scol0 + c])
                plsc.store_scatter(outbuf[b], [qrow, dcol0 + c], vals)

    g_start(0, 0)
    g_start(1, 1)

    def body(gi, _):
        i0 = gi * 2
        for b in range(2):
            i = i0 + b
            g_wait(b)
            compact(i, b)
            g_start(i + 2, b)
            s_start(i, b)
            s_wait(b)
        return 0

    lax.fori_loop(0, _NCHUNK // 2 - 1, body, 0)

    for b in range(2):
        g_wait(b)
        compact(_NCHUNK - 2 + b, b)
        s_start(_NCHUNK - 2 + b, b)
        s_wait(b)


@jax.jit
def _embed_lookup(idx2, table_w):
    mesh = plsc.VectorSubcoreMesh(core_axis_name="c", subcore_axis_name="s")
    grab = pl.kernel(
        _gather_body,
        out_type=jax.ShapeDtypeStruct((_B // _ROWS_PER_LINE, 128), jnp.float32),
        mesh=mesh,
        scratch_types=(
            [
                pltpu.VMEM((_NCHUNK, _CHUNK), jnp.int32),
                pltpu.VMEM((_NCHUNK, _CHUNK), jnp.int32),
                pltpu.VMEM((_CHUNK, 128), jnp.float32),
                pltpu.VMEM((_CHUNK, 128), jnp.float32),
                pltpu.VMEM((_OLINES, 128), jnp.float32),
                pltpu.VMEM((_OLINES, 128), jnp.float32),
            ]
            + [pltpu.SemaphoreType.DMA] * 4
        ),
        compiler_params=pltpu.CompilerParams(needs_layout_passes=False),
    )
    return grab(idx2, table_w)


def kernel(severity_ids, table):
    idx2 = severity_ids.astype(jnp.int32).reshape(_B // _CHUNK, _CHUNK)
    table_w = table.reshape(NUM_CLASSES // _ROWS_PER_LINE, 128)
    out_w = _embed_lookup(idx2, table_w)
    return out_w.reshape(BATCH, FIELDS, EMBED_DIM)


# speedup vs baseline: 1.0172x; 1.0172x over previous
import functools

import jax
import jax.numpy as jnp
from jax import lax
from jax.experimental import pallas as pl
from jax.experimental.pallas import tpu as pltpu
from jax.experimental.pallas import tpu_sc as plsc

NUM_CLASSES = 1000000
EMBED_DIM = 32
BATCH = 16384
FIELDS = 26

_B = BATCH * FIELDS
_NC = 2
_NS = 16
_NW = _NC * _NS
_PER_W = _B // _NW
_CHUNK = 128
_NCHUNK = _PER_W // _CHUNK
_ROWS_PER_LINE = 128 // EMBED_DIM
_OLINES = _CHUNK // _ROWS_PER_LINE
assert _PER_W * _NW == _B and _NCHUNK * _CHUNK == _PER_W
assert _NCHUNK % 2 == 0


def _gather_body(idx_hbm, table_hbm, out_hbm,
                 idxs, lines, line0, line1, out0, out1,
                 gsem0, gsem1, ssem0, ssem1):
    linebuf = (line0, line1)
    outbuf = (out0, out1)
    gsem = (gsem0, gsem1)
    ssem = (ssem0, ssem1)

    wid = lax.axis_index("s") * _NC + lax.axis_index("c")
    base_row = wid * _NCHUNK
    obase = wid * (_PER_W // _ROWS_PER_LINE)

    pltpu.sync_copy(idx_hbm.at[pl.ds(base_row, _NCHUNK), :], idxs)

    def conv(r, _):
        for g in range(8):
            v = idxs[r, pl.ds(g * 16, 16)]
            lines[r, pl.ds(g * 16, 16)] = v >> 2
        return 0

    lax.fori_loop(0, _NCHUNK, conv, 0)

    def g_start(i, b):
        pltpu.make_async_copy(
            table_hbm.at[lines.at[i]], linebuf[b], gsem[b]
        ).start()

    def g_wait(b):
        pltpu.make_async_copy(
            table_hbm.at[lines.at[0]], linebuf[b], gsem[b]
        ).wait()

    def s_start(i, b):
        pltpu.make_async_copy(
            outbuf[b], out_hbm.at[pl.ds(obase + i * _OLINES, _OLINES)], ssem[b]
        ).start()

    def s_wait(b):
        pltpu.make_async_copy(
            outbuf[b], out_hbm.at[pl.ds(obase, _OLINES)], ssem[b]
        ).wait()

    iota = lax.iota(jnp.int32, 16)

    def compact(i, b):
        def cg(g, _):
            r0 = pl.multiple_of(g * 16, 16)
            rowv = iota + r0
            qrow = rowv >> 2
            dcol0 = (rowv & 3) * 32
            idxv = idxs[i, pl.ds(r0, 16)]
            scol0 = (idxv & 3) * 32
            for c in range(EMBED_DIM):
                vals = plsc.load_gather(linebuf[b], [rowv, scol0 + c])
                plsc.store_scatter(outbuf[b], [qrow, dcol0 + c], vals)
            return 0

        lax.fori_loop(0, _CHUNK // 16, cg, 0)

    g_start(0, 0)
    g_start(1, 1)

    def body(gi, _):
        i0 = gi * 2
        for b in range(2):
            i = i0 + b
            g_wait(b)
            compact(i, b)
            g_start(i + 2, b)
            s_start(i, b)
            s_wait(b)
        return 0

    lax.fori_loop(0, _NCHUNK // 2 - 1, body, 0)

    for b in range(2):
        g_wait(b)
        compact(_NCHUNK - 2 + b, b)
        s_start(_NCHUNK - 2 + b, b)
        s_wait(b)


@jax.jit
def _embed_lookup(idx2, table_w):
    mesh = plsc.VectorSubcoreMesh(core_axis_name="c", subcore_axis_name="s")
    grab = pl.kernel(
        _gather_body,
        out_type=jax.ShapeDtypeStruct((_B // _ROWS_PER_LINE, 128), jnp.float32),
        mesh=mesh,
        scratch_types=(
            [
                pltpu.VMEM((_NCHUNK, _CHUNK), jnp.int32),
                pltpu.VMEM((_NCHUNK, _CHUNK), jnp.int32),
                pltpu.VMEM((_CHUNK, 128), jnp.float32),
                pltpu.VMEM((_CHUNK, 128), jnp.float32),
                pltpu.VMEM((_OLINES, 128), jnp.float32),
                pltpu.VMEM((_OLINES, 128), jnp.float32),
            ]
            + [pltpu.SemaphoreType.DMA] * 4
        ),
        compiler_params=pltpu.CompilerParams(needs_layout_passes=False),
    )
    return grab(idx2, table_w)


def kernel(severity_ids, table):
    idx2 = severity_ids.astype(jnp.int32).reshape(_B // _CHUNK, _CHUNK)
    table_w = table.reshape(NUM_CLASSES // _ROWS_PER_LINE, 128)
    out_w = _embed_lookup(idx2, table_w)
    return out_w.reshape(BATCH, FIELDS, EMBED_DIM)


# speedup vs baseline: 1.1285x; 1.1094x over previous
import functools

import jax
import jax.numpy as jnp
from jax import lax
from jax.experimental import pallas as pl
from jax.experimental.pallas import tpu as pltpu
from jax.experimental.pallas import tpu_sc as plsc

NUM_CLASSES = 1000000
EMBED_DIM = 32
BATCH = 16384
FIELDS = 26

_B = BATCH * FIELDS
_NC = 2
_NS = 16
_NW = _NC * _NS
_PER_W = _B // _NW
_CHUNK = 128
_NCHUNK = _PER_W // _CHUNK
_ROWS_PER_LINE = 128 // EMBED_DIM
_OLINES = _CHUNK // _ROWS_PER_LINE
assert _PER_W * _NW == _B and _NCHUNK * _CHUNK == _PER_W
assert _NCHUNK % 2 == 0


def _gather_body(idx_hbm, table_hbm, out_hbm,
                 idxs, lines, line0, line1, out0, out1,
                 gsem0, gsem1, ssem0, ssem1):
    linebuf = (line0, line1)
    outbuf = (out0, out1)
    gsem = (gsem0, gsem1)
    ssem = (ssem0, ssem1)

    wid = lax.axis_index("s") * _NC + lax.axis_index("c")
    base_row = wid * _NCHUNK
    obase = wid * (_PER_W // _ROWS_PER_LINE)

    pltpu.sync_copy(idx_hbm.at[pl.ds(base_row, _NCHUNK), :], idxs)

    def conv(r, _):
        for g in range(8):
            v = idxs[r, pl.ds(g * 16, 16)]
            lines[r, pl.ds(g * 16, 16)] = v >> 2
        return 0

    lax.fori_loop(0, _NCHUNK, conv, 0)

    def g_start(i, b):
        pltpu.make_async_copy(
            table_hbm.at[lines.at[i]], linebuf[b], gsem[b]
        ).start()

    def g_wait(b):
        pltpu.make_async_copy(
            table_hbm.at[lines.at[0]], linebuf[b], gsem[b]
        ).wait()

    def s_start(i, b):
        pltpu.make_async_copy(
            outbuf[b], out_hbm.at[pl.ds(obase + i * _OLINES, _OLINES)], ssem[b]
        ).start()

    def s_wait(b):
        pltpu.make_async_copy(
            outbuf[b], out_hbm.at[pl.ds(obase, _OLINES)], ssem[b]
        ).wait()

    iota = lax.iota(jnp.int32, 16)

    def compact(i, b):
        @plsc.parallel_loop(0, _CHUNK // 16, unroll=2)
        def cg(g):
            r0 = pl.multiple_of(g * 16, 16)
            rowv = iota + r0
            qrow = rowv >> 2
            dcol0 = (rowv & 3) * 32
            idxv = idxs[i, pl.ds(r0, 16)]
            scol0 = (idxv & 3) * 32
            for c in range(EMBED_DIM):
                vals = plsc.load_gather(linebuf[b], [rowv, scol0 + c])
                plsc.store_scatter(outbuf[b], [qrow, dcol0 + c], vals)

    g_start(0, 0)
    g_start(1, 1)

    def body(gi, _):
        i0 = gi * 2
        for b in range(2):
            i = i0 + b
            g_wait(b)
            compact(i, b)
            g_start(i + 2, b)
            s_start(i, b)
            s_wait(b)
        return 0

    lax.fori_loop(0, _NCHUNK // 2 - 1, body, 0)

    for b in range(2):
        g_wait(b)
        compact(_NCHUNK - 2 + b, b)
        s_start(_NCHUNK - 2 + b, b)
        s_wait(b)


@jax.jit
def _embed_lookup(idx2, table_w):
    mesh = plsc.VectorSubcoreMesh(core_axis_name="c", subcore_axis_name="s")
    grab = pl.kernel(
        _gather_body,
        out_type=jax.ShapeDtypeStruct((_B // _ROWS_PER_LINE, 128), jnp.float32),
        mesh=mesh,
        scratch_types=(
            [
                pltpu.VMEM((_NCHUNK, _CHUNK), jnp.int32),
                pltpu.VMEM((_NCHUNK, _CHUNK), jnp.int32),
                pltpu.VMEM((_CHUNK, 128), jnp.float32),
                pltpu.VMEM((_CHUNK, 128), jnp.float32),
                pltpu.VMEM((_OLINES, 128), jnp.float32),
                pltpu.VMEM((_OLINES, 128), jnp.float32),
            ]
            + [pltpu.SemaphoreType.DMA] * 4
        ),
        compiler_params=pltpu.CompilerParams(needs_layout_passes=False),
    )
    return grab(idx2, table_w)


def kernel(severity_ids, table):
    idx2 = severity_ids.astype(jnp.int32).reshape(_B // _CHUNK, _CHUNK)
    table_w = table.reshape(NUM_CLASSES // _ROWS_PER_LINE, 128)
    out_w = _embed_lookup(idx2, table_w)
    return out_w.reshape(BATCH, FIELDS, EMBED_DIM)


# speedup vs baseline: 1.4475x; 1.2827x over previous
import functools

import jax
import jax.numpy as jnp
from jax import lax
from jax.experimental import pallas as pl
from jax.experimental.pallas import tpu as pltpu
from jax.experimental.pallas import tpu_sc as plsc

NUM_CLASSES = 1000000
EMBED_DIM = 32
BATCH = 16384
FIELDS = 26

_B = BATCH * FIELDS
_NC = 2
_NS = 16
_NW = _NC * _NS
_PER_W = _B // _NW
_BROW = 8
_CHUNK = _BROW * FIELDS
_NCHUNK = _PER_W // _CHUNK
_RPL = 128 // EMBED_DIM
assert _PER_W * _NW == _B and _NCHUNK * _CHUNK == _PER_W
assert _NCHUNK % 2 == 0 and _CHUNK % 8 == 0


def _gather_body(idx_hbm, table_hbm, out_hbm,
                 idx_v, lin0, lin1, rows0, rows1, stage,
                 gsem0, gsem1, ssem0, ssem1):
    rows = (rows0, rows1)
    linc = (lin0, lin1)
    gsem = (gsem0, gsem1)
    ssem = (ssem0, ssem1)

    wid = lax.axis_index("s") * _NC + lax.axis_index("c")
    base = wid * _PER_W
    brow0 = wid * (_PER_W // FIELDS)
    pltpu.sync_copy(idx_hbm.at[pl.ds(base, _PER_W)], idx_v)

    def g_start(i, b):
        coff = i * _CHUNK
        for k in range(_CHUNK // 16):
            linc[b][pl.ds(k * 16, 16)] = (
                idx_v[pl.ds(coff + k * 16, 16)] >> 2
            )
        pltpu.make_async_copy(
            table_hbm.at[linc[b]], rows[b], gsem[b]
        ).start()

    def g_wait(b):
        pltpu.make_async_copy(
            table_hbm.at[linc[b]], rows[b], gsem[b]
        ).wait()

    def s_start(i, b):
        pltpu.make_async_copy(
            stage, out_hbm.at[pl.ds(brow0 + i * _BROW, _BROW)], ssem[b]
        ).start()

    def s_wait(b):
        pltpu.make_async_copy(
            stage, out_hbm.at[pl.ds(brow0, _BROW)], ssem[b]
        ).wait()

    def compact(i, b):
        coff = i * _CHUNK
        for k in range(_CHUNK // 16):
            v = idx_v[pl.ds(coff + k * 16, 16)]
            subs = (v & 3) * 32
            for l in range(16):
                r = k * 16 + l
                bi, f = divmod(r, FIELDS)
                scol = subs[l]
                for j in range(EMBED_DIM // 16):
                    stage[bi, f, pl.ds(j * 16, 16)] = (
                        rows[b][r, pl.ds(scol + j * 16, 16)]
                    )

    g_start(0, 0)
    g_start(1, 1)

    def body(gi, _):
        i0 = gi * 2
        for b in range(2):
            i = i0 + b
            g_wait(b)
            compact(i, b)
            g_start(i + 2, b)
            s_start(i, b)
            s_wait(b)
        return 0

    lax.fori_loop(0, _NCHUNK // 2 - 1, body, 0)

    for b in range(2):
        g_wait(b)
        compact(_NCHUNK - 2 + b, b)
        s_start(_NCHUNK - 2 + b, b)
        s_wait(b)


@jax.jit
def _embed_lookup(idx_flat, table_w):
    mesh = plsc.VectorSubcoreMesh(core_axis_name="c", subcore_axis_name="s")
    grab = pl.kernel(
        _gather_body,
        out_type=jax.ShapeDtypeStruct((BATCH, FIELDS, EMBED_DIM), jnp.float32),
        mesh=mesh,
        scratch_types=(
            [pltpu.VMEM((_PER_W,), jnp.int32)]
            + [pltpu.VMEM((_CHUNK,), jnp.int32)] * 2
            + [pltpu.VMEM((_CHUNK, 128), jnp.float32)] * 2
            + [pltpu.VMEM((_BROW, FIELDS, EMBED_DIM), jnp.float32)]
            + [pltpu.SemaphoreType.DMA] * 4
        ),
    )
    return grab(idx_flat, table_w)


def kernel(severity_ids, table):
    idx_flat = severity_ids.reshape(_B).astype(jnp.int32)
    table_w = table.reshape(NUM_CLASSES // _RPL, 128)
    return _embed_lookup(idx_flat, table_w)


# speedup vs baseline: 1.6429x; 1.1350x over previous
import functools

import jax
import jax.numpy as jnp
from jax import lax
from jax.experimental import pallas as pl
from jax.experimental.pallas import tpu as pltpu
from jax.experimental.pallas import tpu_sc as plsc

NUM_CLASSES = 1000000
EMBED_DIM = 32
BATCH = 16384
FIELDS = 26

_B = BATCH * FIELDS
_NC = 2
_NS = 16
_NW = _NC * _NS
_PER_W = _B // _NW
_CHUNK = 416
_NCHUNK = _PER_W // _CHUNK
_NBUF = 4
assert _PER_W * _NW == _B and _NCHUNK * _CHUNK == _PER_W
assert _NCHUNK % _NBUF == 0 and _CHUNK % 8 == 0


def _gather_body(idx_hbm, table_hbm, out_hbm, idx_v, *bufs):
    rows = bufs[:_NBUF]
    gsem = bufs[_NBUF:2 * _NBUF]
    ssem = bufs[2 * _NBUF:]

    wid = lax.axis_index("s") * _NC + lax.axis_index("c")
    base = wid * _PER_W
    pltpu.sync_copy(idx_hbm.at[pl.ds(base, _PER_W)], idx_v)

    def g_start(i, b):
        pltpu.make_async_copy(
            table_hbm.at[idx_v.at[pl.ds(i * _CHUNK, _CHUNK)]], rows[b], gsem[b]
        ).start()

    def g_wait(b):
        pltpu.make_async_copy(
            table_hbm.at[idx_v.at[pl.ds(0, _CHUNK)]], rows[b], gsem[b]
        ).wait()

    def s_start(i, b):
        pltpu.make_async_copy(
            rows[b], out_hbm.at[pl.ds(base + i * _CHUNK, _CHUNK)], ssem[b]
        ).start()

    def s_wait(b):
        pltpu.make_async_copy(
            rows[b], out_hbm.at[pl.ds(base, _CHUNK)], ssem[b]
        ).wait()

    for b in range(_NBUF):
        g_start(b, b)

    def body(gi, _):
        i0 = gi * _NBUF
        for b in range(_NBUF):
            i = i0 + b
            g_wait(b)
            s_start(i, b)
            s_wait(b)
            g_start(i + _NBUF, b)
        return 0

    lax.fori_loop(0, _NCHUNK // _NBUF - 1, body, 0)

    for b in range(_NBUF):
        g_wait(b)
        s_start(_NCHUNK - _NBUF + b, b)
        s_wait(b)


@jax.jit
def _embed_lookup(idx_flat, table):
    mesh = plsc.VectorSubcoreMesh(core_axis_name="c", subcore_axis_name="s")
    grab = pl.kernel(
        _gather_body,
        out_type=jax.ShapeDtypeStruct((_B, EMBED_DIM), jnp.float32),
        mesh=mesh,
        scratch_types=(
            [pltpu.VMEM((_PER_W,), jnp.int32)]
            + [pltpu.VMEM((_CHUNK, EMBED_DIM), jnp.float32)] * _NBUF
            + [pltpu.SemaphoreType.DMA] * (2 * _NBUF)
        ),
        compiler_params=pltpu.CompilerParams(use_tc_tiling_on_sc=False),
    )
    return grab(idx_flat, table)


def kernel(severity_ids, table):
    idx_flat = severity_ids.reshape(_B).astype(jnp.int32)
    out = _embed_lookup(idx_flat, table)
    return out.reshape(BATCH, FIELDS, EMBED_DIM)
